# Initial kernel scaffold; baseline (speedup 1.0000x reference)
import functools

import jax
import jax.numpy as jnp
from jax import lax
from jax.experimental import pallas as pl
from jax.experimental.pallas import tpu as pltpu
from jax.experimental.pallas import tpu_sc as plsc

NC, NS, L = 2, 16, 16
NW = NC * NS
CH = 128


@functools.lru_cache(maxsize=None)
def _gather2(N, D, E):
    EPW = E // NW
    NFULL = EPW // CH
    TAIL = EPW - NFULL * CH
    mesh = plsc.VectorSubcoreMesh(core_axis_name="c", subcore_axis_name="s")

    @functools.partial(
        pl.kernel,
        mesh=mesh,
        out_type=(
            jax.ShapeDtypeStruct((E, D), jnp.float32),
            jax.ShapeDtypeStruct((E, D), jnp.float32),
        ),
        scratch_types=[
            pltpu.VMEM((CH,), jnp.int32),
            pltpu.VMEM((CH, D), jnp.float32),
            pltpu.SemaphoreType.DMA,
        ],
    )
    def k(table_hbm, row_hbm, col_hbm, xr_hbm, xc_hbm, idx_v, rows_v, sem):
        wid = lax.axis_index("s") * NC + lax.axis_index("c")
        base = wid * EPW

        def do_chunk(idx_hbm, out_hbm, off, sz):
            pltpu.sync_copy(idx_hbm.at[pl.ds(off, sz)], idx_v.at[pl.ds(0, sz)])
            pltpu.async_copy(
                table_hbm.at[idx_v.at[pl.ds(0, sz)]],
                rows_v.at[pl.ds(0, sz)],
                sem,
            ).wait()
            pltpu.sync_copy(rows_v.at[pl.ds(0, sz)], out_hbm.at[pl.ds(off, sz)])

        def do(idx_hbm, out_hbm):
            @pl.loop(0, NFULL)
            def _(c):
                off = pl.multiple_of(base + c * CH, 8)
                do_chunk(idx_hbm, out_hbm, off, CH)

            if TAIL:
                do_chunk(idx_hbm, out_hbm, base + NFULL * CH, TAIL)

        do(row_hbm, xr_hbm)
        do(col_hbm, xc_hbm)

    return k


@functools.lru_cache(maxsize=None)
def _scatter_add(E, D, N):
    EPW = E // NW
    NFULL = EPW // CH
    TAIL = EPW - NFULL * CH
    RPT = N // NS
    mesh = plsc.VectorSubcoreMesh(core_axis_name="c", subcore_axis_name="s")

    @functools.partial(
        pl.kernel,
        mesh=mesh,
        out_type=jax.ShapeDtypeStruct((NC, N, D), jnp.float32),
        scratch_types=[
            pltpu.VMEM((1, CH), jnp.int32),
            pltpu.VMEM((CH, D), jnp.float32),
            pltpu.VMEM_SHARED((N, D), jnp.float32),
        ],
    )
    def k(vals_hbm, idx_hbm, zeros_hbm, out_hbm, idx_v, vals_v, acc):
        cid = lax.axis_index("c")
        sid = lax.axis_index("s")
        wid = sid * NC + cid
        base = wid * EPW

        pltpu.sync_copy(
            zeros_hbm.at[pl.ds(sid * RPT, RPT)], acc.at[pl.ds(sid * RPT, RPT)]
        )
        plsc.subcore_barrier()

        def do_chunk(off, sz):
            pltpu.sync_copy(idx_hbm.at[pl.ds(off, sz)], idx_v.at[0, pl.ds(0, sz)])
            pltpu.sync_copy(vals_hbm.at[pl.ds(off, sz)], vals_v.at[pl.ds(0, sz)])
            pltpu.sync_copy(
                vals_v.at[pl.ds(0, sz)],
                acc.at[idx_v.at[0, pl.ds(0, sz)]],
                add=True,
            )

        @pl.loop(0, NFULL)
        def _(c):
            do_chunk(pl.multiple_of(base + c * CH, 8), CH)

        if TAIL:
            do_chunk(base + NFULL * CH, TAIL)

        plsc.subcore_barrier()
        pltpu.sync_copy(
            acc.at[pl.ds(sid * RPT, RPT)], out_hbm.at[cid, pl.ds(sid * RPT, RPT)]
        )

    return k


_BE = 4000
_BN = 2000


def _full(shape):
    return pl.BlockSpec(shape, lambda i: (0,) * len(shape))


def _edge_body(xr_r, xc_r, ea_r, A, B, C, b1, W2, b2, D1, Ew, b1n, W2n, b2n,
               ea2_r, h_r):
    f32 = jnp.float32
    t = xr_r[...] @ A[...] + xc_r[...] @ B[...] + ea_r[...] @ C[...] + b1[...]
    t = jnp.maximum(t, 0.0)
    ea2 = jax.lax.dot(t, W2[...], preferred_element_type=f32) + b2[...]
    ea2_r[...] = ea2
    t2 = xr_r[...] @ D1[...] + ea2 @ Ew[...] + b1n[...]
    t2 = jnp.maximum(t2, 0.0)
    h_r[...] = jax.lax.dot(t2, W2n[...], preferred_element_type=f32) + b2n[...]


def _tc_edge(xr, xc, ea, A, B, C, b1, W2, b2, D1, Ew, b1n, W2n, b2n):
    E, Dx = xr.shape
    De = ea.shape[1]
    G = E // _BE
    grid_spec = pl.GridSpec(
        grid=(G,),
        in_specs=[
            pl.BlockSpec((_BE, Dx), lambda i: (i, 0)),
            pl.BlockSpec((_BE, Dx), lambda i: (i, 0)),
            pl.BlockSpec((_BE, De), lambda i: (i, 0)),
            _full(A.shape), _full(B.shape), _full(C.shape), _full(b1.shape),
            _full(W2.shape), _full(b2.shape), _full(D1.shape), _full(Ew.shape),
            _full(b1n.shape), _full(W2n.shape), _full(b2n.shape),
        ],
        out_specs=[
            pl.BlockSpec((_BE, 64), lambda i: (i, 0)),
            pl.BlockSpec((_BE, 64), lambda i: (i, 0)),
        ],
    )
    return pl.pallas_call(
        _edge_body,
        grid_spec=grid_spec,
        out_shape=[
            jax.ShapeDtypeStruct((E, 64), jnp.float32),
            jax.ShapeDtypeStruct((E, 64), jnp.float32),
        ],
        compiler_params=pltpu.CompilerParams(
            dimension_semantics=("arbitrary",)
        ),
    )(xr, xc, ea, A, B, C, b1, W2, b2, D1, Ew, b1n, W2n, b2n)


def _edge_final_body(xr_r, xc_r, ea_r, A, B, C, b1, W2, b2, out_r):
    t = xr_r[...] @ A[...] + xc_r[...] @ B[...] + ea_r[...] @ C[...] + b1[...]
    t = jnp.maximum(t, 0.0)
    z = jax.lax.dot(t, W2[...], preferred_element_type=jnp.float32) + b2[...]
    out_r[...] = jax.nn.sigmoid(z)


def _tc_edge_final(xr, xc, ea, A, B, C, b1, W2, b2):
    E, Dx = xr.shape
    De = ea.shape[1]
    G = E // _BE
    grid_spec = pl.GridSpec(
        grid=(G,),
        in_specs=[
            pl.BlockSpec((_BE, Dx), lambda i: (i, 0)),
            pl.BlockSpec((_BE, Dx), lambda i: (i, 0)),
            pl.BlockSpec((_BE, De), lambda i: (i, 0)),
            _full(A.shape), _full(B.shape), _full(C.shape), _full(b1.shape),
            _full(W2.shape), _full(b2.shape),
        ],
        out_specs=[pl.BlockSpec((_BE, 1), lambda i: (i, 0))],
    )
    return pl.pallas_call(
        _edge_final_body,
        grid_spec=grid_spec,
        out_shape=[jax.ShapeDtypeStruct((E, 1), jnp.float32)],
        compiler_params=pltpu.CompilerParams(
            dimension_semantics=("arbitrary",)
        ),
    )(xr, xc, ea, A, B, C, b1, W2, b2)[0]


def _node_body(x_r, sp_r, cp_r, Wx, Wa, b1, W2, b2, out_r):
    s = sp_r[0] + sp_r[1]
    c = cp_r[0, :, 0:1] + cp_r[1, :, 0:1]
    agg = s / jnp.maximum(c, 1.0)
    t = x_r[...] @ Wx[...] + agg @ Wa[...] + b1[...]
    t = jnp.maximum(t, 0.0)
    out_r[...] = jax.lax.dot(t, W2[...], preferred_element_type=jnp.float32) + b2[...]


def _tc_node(x, spart, cpart, Wx, Wa, b1, W2, b2):
    N, Dx = x.shape
    G = N // _BN
    grid_spec = pl.GridSpec(
        grid=(G,),
        in_specs=[
            pl.BlockSpec((_BN, Dx), lambda i: (i, 0)),
            pl.BlockSpec((NC, _BN, 64), lambda i: (0, i, 0)),
            pl.BlockSpec((NC, _BN, 16), lambda i: (0, i, 0)),
            _full(Wx.shape), _full(Wa.shape), _full(b1.shape),
            _full(W2.shape), _full(b2.shape),
        ],
        out_specs=[pl.BlockSpec((_BN, 64), lambda i: (i, 0))],
    )
    return pl.pallas_call(
        _node_body,
        grid_spec=grid_spec,
        out_shape=[jax.ShapeDtypeStruct((N, 64), jnp.float32)],
        compiler_params=pltpu.CompilerParams(
            dimension_semantics=("arbitrary",)
        ),
    )(x, spart, cpart, Wx, Wa, b1, W2, b2)[0]


def _row(b):
    return b.reshape(1, -1)


def _pad_rows(W, rows_from, n_rows):
    sub = W[rows_from[0]:rows_from[1]]
    return jnp.pad(sub, ((0, n_rows - sub.shape[0]), (0, 0)))


def kernel(x, edge_index, edge_attr, params):
    N, _ = x.shape
    E = edge_attr.shape[0]
    row, col = edge_index[0], edge_index[1]

    x16 = jnp.pad(x, ((0, 0), (0, 14)))
    ea16 = jnp.pad(edge_attr, ((0, 0), (0, 12)))
    zeros64 = jnp.zeros((N, 64), jnp.float32)
    zeros16 = jnp.zeros((N, 16), jnp.float32)
    ones16 = jnp.ones((E, 16), jnp.float32)

    cpart = _scatter_add(E, 16, N)(ones16, col, zeros16)

    W1e, b1e, W2e, b2e = params['c1_e']
    W1n, b1n, W2n, b2n = params['c1_n1']
    V1n, c1n, V2n, c2n = params['c1_n2']
    A = _pad_rows(W1e, (0, 2), 16)
    B = _pad_rows(W1e, (2, 4), 16)
    C = _pad_rows(W1e, (4, 8), 16)
    D1 = _pad_rows(W1n, (0, 2), 16)
    Ew = W1n[2:66]
    Wx = _pad_rows(V1n, (0, 2), 16)
    Wa = V1n[2:66]

    xr, xc = _gather2(N, 16, E)(x16, row, col)
    ea, h = _tc_edge(xr, xc, ea16, A, B, C, _row(b1e), W2e, _row(b2e),
                     D1, Ew, _row(b1n), W2n, _row(b2n))
    spart = _scatter_add(E, 64, N)(h, col, zeros64)
    xl = _tc_node(x16, spart, cpart, Wx, Wa, _row(c1n), V2n, _row(c2n))

    for name in ('c2', 'c3', 'c4'):
        W1e, b1e, W2e, b2e = params[name + '_e']
        W1n, b1n, W2n, b2n = params[name + '_n1']
        V1n, c1n, V2n, c2n = params[name + '_n2']
        xr, xc = _gather2(N, 64, E)(xl, row, col)
        ea, h = _tc_edge(xr, xc, ea,
                         W1e[0:64], W1e[64:128], W1e[128:192], _row(b1e),
                         W2e, _row(b2e),
                         W1n[0:64], W1n[64:128], _row(b1n), W2n, _row(b2n))
        spart = _scatter_add(E, 64, N)(h, col, zeros64)
        xl = _tc_node(xl, spart, cpart,
                      V1n[0:64], V1n[64:128], _row(c1n), V2n, _row(c2n))

    W1f, b1f, W2f, b2f = params['c5_e']
    xr, xc = _gather2(N, 64, E)(xl, row, col)
    return _tc_edge_final(xr, xc, ea,
                          W1f[0:64], W1f[64:128], W1f[128:192], _row(b1f),
                          W2f, _row(b2f))


# speedup vs baseline: 1.8365x; 1.8365x over previous
import functools

import jax
import jax.numpy as jnp
from jax import lax
from jax.experimental import pallas as pl
from jax.experimental.pallas import tpu as pltpu
from jax.experimental.pallas import tpu_sc as plsc

NC, NS, L = 2, 16, 16
NW = NC * NS
CH = 128


@functools.lru_cache(maxsize=None)
def _gather2(N, D, E):
    EPW = E // NW
    NFULL = EPW // CH
    TAIL = EPW - NFULL * CH
    mesh = plsc.VectorSubcoreMesh(core_axis_name="c", subcore_axis_name="s")

    @functools.partial(
        pl.kernel,
        mesh=mesh,
        out_type=(
            jax.ShapeDtypeStruct((E, D), jnp.float32),
            jax.ShapeDtypeStruct((E, D), jnp.float32),
        ),
        scratch_types=[
            pltpu.VMEM((CH,), jnp.int32),
            pltpu.VMEM((CH, D), jnp.float32),
            pltpu.SemaphoreType.DMA,
        ],
        compiler_params=pltpu.CompilerParams(use_tc_tiling_on_sc=False),
    )
    def k(table_hbm, row_hbm, col_hbm, xr_hbm, xc_hbm, idx_v, rows_v, sem):
        wid = lax.axis_index("s") * NC + lax.axis_index("c")
        base = wid * EPW

        def do_chunk(idx_hbm, out_hbm, off, sz):
            pltpu.sync_copy(idx_hbm.at[pl.ds(off, sz)], idx_v.at[pl.ds(0, sz)])
            pltpu.async_copy(
                table_hbm.at[idx_v.at[pl.ds(0, sz)]],
                rows_v.at[pl.ds(0, sz)],
                sem,
            ).wait()
            pltpu.sync_copy(rows_v.at[pl.ds(0, sz)], out_hbm.at[pl.ds(off, sz)])

        def do(idx_hbm, out_hbm):
            @pl.loop(0, NFULL)
            def _(c):
                off = pl.multiple_of(base + c * CH, 8)
                do_chunk(idx_hbm, out_hbm, off, CH)

            if TAIL:
                do_chunk(idx_hbm, out_hbm, base + NFULL * CH, TAIL)

        do(row_hbm, xr_hbm)
        do(col_hbm, xc_hbm)

    return k


@functools.lru_cache(maxsize=None)
def _scatter_add(E, D, N):
    EPW = E // NW
    NFULL = EPW // CH
    TAIL = EPW - NFULL * CH
    RPT = N // NS
    mesh = plsc.VectorSubcoreMesh(core_axis_name="c", subcore_axis_name="s")

    @functools.partial(
        pl.kernel,
        mesh=mesh,
        out_type=jax.ShapeDtypeStruct((NC, N, D), jnp.float32),
        scratch_types=[
            pltpu.VMEM((1, CH), jnp.int32),
            pltpu.VMEM((CH, D), jnp.float32),
            pltpu.VMEM_SHARED((N, D), jnp.float32),
        ],
        compiler_params=pltpu.CompilerParams(use_tc_tiling_on_sc=False),
    )
    def k(vals_hbm, idx_hbm, zeros_hbm, out_hbm, idx_v, vals_v, acc):
        cid = lax.axis_index("c")
        sid = lax.axis_index("s")
        wid = sid * NC + cid
        base = wid * EPW

        pltpu.sync_copy(
            zeros_hbm.at[pl.ds(sid * RPT, RPT)], acc.at[pl.ds(sid * RPT, RPT)]
        )
        plsc.subcore_barrier()

        def do_chunk(off, sz):
            pltpu.sync_copy(idx_hbm.at[pl.ds(off, sz)], idx_v.at[0, pl.ds(0, sz)])
            pltpu.sync_copy(vals_hbm.at[pl.ds(off, sz)], vals_v.at[pl.ds(0, sz)])
            pltpu.sync_copy(
                vals_v.at[pl.ds(0, sz)],
                acc.at[idx_v.at[0, pl.ds(0, sz)]],
                add=True,
            )

        @pl.loop(0, NFULL)
        def _(c):
            do_chunk(pl.multiple_of(base + c * CH, 8), CH)

        if TAIL:
            do_chunk(base + NFULL * CH, TAIL)

        plsc.subcore_barrier()
        pltpu.sync_copy(
            acc.at[pl.ds(sid * RPT, RPT)], out_hbm.at[cid, pl.ds(sid * RPT, RPT)]
        )

    return k


_BE = 4000
_BN = 2000


def _full(shape):
    return pl.BlockSpec(shape, lambda i: (0,) * len(shape))


def _edge_body(xr_r, xc_r, ea_r, A, B, C, b1, W2, b2, D1, Ew, b1n, W2n, b2n,
               ea2_r, h_r):
    f32 = jnp.float32
    t = xr_r[...] @ A[...] + xc_r[...] @ B[...] + ea_r[...] @ C[...] + b1[...]
    t = jnp.maximum(t, 0.0)
    ea2 = jax.lax.dot(t, W2[...], preferred_element_type=f32) + b2[...]
    ea2_r[...] = ea2
    t2 = xr_r[...] @ D1[...] + ea2 @ Ew[...] + b1n[...]
    t2 = jnp.maximum(t2, 0.0)
    h_r[...] = jax.lax.dot(t2, W2n[...], preferred_element_type=f32) + b2n[...]


def _tc_edge(xr, xc, ea, A, B, C, b1, W2, b2, D1, Ew, b1n, W2n, b2n):
    E, Dx = xr.shape
    De = ea.shape[1]
    G = E // _BE
    grid_spec = pl.GridSpec(
        grid=(G,),
        in_specs=[
            pl.BlockSpec((_BE, Dx), lambda i: (i, 0)),
            pl.BlockSpec((_BE, Dx), lambda i: (i, 0)),
            pl.BlockSpec((_BE, De), lambda i: (i, 0)),
            _full(A.shape), _full(B.shape), _full(C.shape), _full(b1.shape),
            _full(W2.shape), _full(b2.shape), _full(D1.shape), _full(Ew.shape),
            _full(b1n.shape), _full(W2n.shape), _full(b2n.shape),
        ],
        out_specs=[
            pl.BlockSpec((_BE, 64), lambda i: (i, 0)),
            pl.BlockSpec((_BE, 64), lambda i: (i, 0)),
        ],
    )
    return pl.pallas_call(
        _edge_body,
        grid_spec=grid_spec,
        out_shape=[
            jax.ShapeDtypeStruct((E, 64), jnp.float32),
            jax.ShapeDtypeStruct((E, 64), jnp.float32),
        ],
        compiler_params=pltpu.CompilerParams(
            dimension_semantics=("arbitrary",)
        ),
    )(xr, xc, ea, A, B, C, b1, W2, b2, D1, Ew, b1n, W2n, b2n)


def _edge_final_body(xr_r, xc_r, ea_r, A, B, C, b1, W2, b2, out_r):
    t = xr_r[...] @ A[...] + xc_r[...] @ B[...] + ea_r[...] @ C[...] + b1[...]
    t = jnp.maximum(t, 0.0)
    z = jax.lax.dot(t, W2[...], preferred_element_type=jnp.float32) + b2[...]
    out_r[...] = jax.nn.sigmoid(z)


def _tc_edge_final(xr, xc, ea, A, B, C, b1, W2, b2):
    E, Dx = xr.shape
    De = ea.shape[1]
    G = E // _BE
    grid_spec = pl.GridSpec(
        grid=(G,),
        in_specs=[
            pl.BlockSpec((_BE, Dx), lambda i: (i, 0)),
            pl.BlockSpec((_BE, Dx), lambda i: (i, 0)),
            pl.BlockSpec((_BE, De), lambda i: (i, 0)),
            _full(A.shape), _full(B.shape), _full(C.shape), _full(b1.shape),
            _full(W2.shape), _full(b2.shape),
        ],
        out_specs=[pl.BlockSpec((_BE, 1), lambda i: (i, 0))],
    )
    return pl.pallas_call(
        _edge_final_body,
        grid_spec=grid_spec,
        out_shape=[jax.ShapeDtypeStruct((E, 1), jnp.float32)],
        compiler_params=pltpu.CompilerParams(
            dimension_semantics=("arbitrary",)
        ),
    )(xr, xc, ea, A, B, C, b1, W2, b2)[0]


def _node_body(x_r, sp_r, cp_r, Wx, Wa, b1, W2, b2, out_r):
    s = sp_r[0] + sp_r[1]
    c = cp_r[0, :, 0:1] + cp_r[1, :, 0:1]
    agg = s / jnp.maximum(c, 1.0)
    t = x_r[...] @ Wx[...] + agg @ Wa[...] + b1[...]
    t = jnp.maximum(t, 0.0)
    out_r[...] = jax.lax.dot(t, W2[...], preferred_element_type=jnp.float32) + b2[...]


def _tc_node(x, spart, cpart, Wx, Wa, b1, W2, b2):
    N, Dx = x.shape
    G = N // _BN
    grid_spec = pl.GridSpec(
        grid=(G,),
        in_specs=[
            pl.BlockSpec((_BN, Dx), lambda i: (i, 0)),
            pl.BlockSpec((NC, _BN, 64), lambda i: (0, i, 0)),
            pl.BlockSpec((NC, _BN, 16), lambda i: (0, i, 0)),
            _full(Wx.shape), _full(Wa.shape), _full(b1.shape),
            _full(W2.shape), _full(b2.shape),
        ],
        out_specs=[pl.BlockSpec((_BN, 64), lambda i: (i, 0))],
    )
    return pl.pallas_call(
        _node_body,
        grid_spec=grid_spec,
        out_shape=[jax.ShapeDtypeStruct((N, 64), jnp.float32)],
        compiler_params=pltpu.CompilerParams(
            dimension_semantics=("arbitrary",)
        ),
    )(x, spart, cpart, Wx, Wa, b1, W2, b2)[0]


def _row(b):
    return b.reshape(1, -1)


def _pad_rows(W, rows_from, n_rows):
    sub = W[rows_from[0]:rows_from[1]]
    return jnp.pad(sub, ((0, n_rows - sub.shape[0]), (0, 0)))


def kernel(x, edge_index, edge_attr, params):
    N, _ = x.shape
    E = edge_attr.shape[0]
    row, col = edge_index[0], edge_index[1]

    x16 = jnp.pad(x, ((0, 0), (0, 14)))
    ea16 = jnp.pad(edge_attr, ((0, 0), (0, 12)))
    zeros64 = jnp.zeros((N, 64), jnp.float32)
    zeros16 = jnp.zeros((N, 16), jnp.float32)
    ones16 = jnp.ones((E, 16), jnp.float32)

    cpart = _scatter_add(E, 16, N)(ones16, col, zeros16)

    W1e, b1e, W2e, b2e = params['c1_e']
    W1n, b1n, W2n, b2n = params['c1_n1']
    V1n, c1n, V2n, c2n = params['c1_n2']
    A = _pad_rows(W1e, (0, 2), 16)
    B = _pad_rows(W1e, (2, 4), 16)
    C = _pad_rows(W1e, (4, 8), 16)
    D1 = _pad_rows(W1n, (0, 2), 16)
    Ew = W1n[2:66]
    Wx = _pad_rows(V1n, (0, 2), 16)
    Wa = V1n[2:66]

    xr, xc = _gather2(N, 16, E)(x16, row, col)
    ea, h = _tc_edge(xr, xc, ea16, A, B, C, _row(b1e), W2e, _row(b2e),
                     D1, Ew, _row(b1n), W2n, _row(b2n))
    spart = _scatter_add(E, 64, N)(h, col, zeros64)
    xl = _tc_node(x16, spart, cpart, Wx, Wa, _row(c1n), V2n, _row(c2n))

    for name in ('c2', 'c3', 'c4'):
        W1e, b1e, W2e, b2e = params[name + '_e']
        W1n, b1n, W2n, b2n = params[name + '_n1']
        V1n, c1n, V2n, c2n = params[name + '_n2']
        xr, xc = _gather2(N, 64, E)(xl, row, col)
        ea, h = _tc_edge(xr, xc, ea,
                         W1e[0:64], W1e[64:128], W1e[128:192], _row(b1e),
                         W2e, _row(b2e),
                         W1n[0:64], W1n[64:128], _row(b1n), W2n, _row(b2n))
        spart = _scatter_add(E, 64, N)(h, col, zeros64)
        xl = _tc_node(xl, spart, cpart,
                      V1n[0:64], V1n[64:128], _row(c1n), V2n, _row(c2n))

    W1f, b1f, W2f, b2f = params['c5_e']
    xr, xc = _gather2(N, 64, E)(xl, row, col)
    return _tc_edge_final(xr, xc, ea,
                          W1f[0:64], W1f[64:128], W1f[128:192], _row(b1f),
                          W2f, _row(b2f))


# speedup vs baseline: 2.3091x; 1.2573x over previous
import functools

import jax
import jax.numpy as jnp
from jax import lax
from jax.experimental import pallas as pl
from jax.experimental.pallas import tpu as pltpu
from jax.experimental.pallas import tpu_sc as plsc

NC, NS, L = 2, 16, 16
NW = NC * NS
CH = 128


GRP = 4


def _tile_rows(E):
    nchk = E // CH
    base = nchk // NW
    extra = nchk - base * NW
    return nchk, base, extra


def _row_start(wid, base, extra):
    return jnp.where(
        wid < extra, (base + 1) * wid, extra * (base + 1) + base * (wid - extra)
    )


@functools.lru_cache(maxsize=None)
def _gather2(N, D, E):
    nchk, base, extra = _tile_rows(E)
    maxrows = base + (1 if extra else 0)
    nfullg = base // GRP
    rem = base - nfullg * GRP
    mesh = plsc.VectorSubcoreMesh(core_axis_name="c", subcore_axis_name="s")

    @functools.partial(
        pl.kernel,
        mesh=mesh,
        out_type=(
            jax.ShapeDtypeStruct((E, D), jnp.float32),
            jax.ShapeDtypeStruct((E, D), jnp.float32),
        ),
        scratch_types=[
            pltpu.VMEM((maxrows, CH), jnp.int32),
            pltpu.VMEM((GRP * CH, D), jnp.float32),
            pltpu.VMEM((GRP * CH, D), jnp.float32),
            pltpu.SemaphoreType.DMA,
            pltpu.SemaphoreType.DMA,
        ],
        compiler_params=pltpu.CompilerParams(use_tc_tiling_on_sc=False),
    )
    def k(table_hbm, row_hbm, col_hbm, xr_hbm, xc_hbm, idx_v, bufa, bufb, gsem, osem):
        wid = lax.axis_index("s") * NC + lax.axis_index("c")
        row0 = _row_start(wid, base, extra)
        has_extra = wid < extra
        bufs = (bufa, bufb)

        def do(idx2d_hbm, out_hbm):
            pltpu.sync_copy(
                idx2d_hbm.at[pl.ds(row0, base)], idx_v.at[pl.ds(0, base)]
            )
            if extra:
                @pl.when(has_extra)
                def _():
                    pltpu.sync_copy(
                        idx2d_hbm.at[pl.ds(row0 + base, 1)],
                        idx_v.at[pl.ds(base, 1)],
                    )

            groups = []
            for g in range(nfullg):
                groups.append((g * GRP, GRP, False))
            if rem or extra:
                groups.append((nfullg * GRP, rem, bool(extra)))

            out_desc = [None, None]
            for gi, (r0, nr, cond) in enumerate(groups):
                buf = bufs[gi % 2]
                if out_desc[gi % 2] is not None:
                    out_desc[gi % 2].wait()
                descs = []
                for j in range(nr):
                    descs.append(
                        pltpu.async_copy(
                            table_hbm.at[idx_v.at[r0 + j]],
                            buf.at[pl.ds(j * CH, CH)],
                            gsem,
                        )
                    )
                if cond:
                    @pl.when(has_extra)
                    def _(r0=r0, nr=nr, buf=buf):
                        pltpu.async_copy(
                            table_hbm.at[idx_v.at[r0 + nr]],
                            buf.at[pl.ds(nr * CH, CH)],
                            gsem,
                        ).wait()
                for d in descs:
                    d.wait()
                off = (row0 + r0) * CH
                out_desc[gi % 2] = pltpu.async_copy(
                    buf.at[pl.ds(0, nr * CH)], out_hbm.at[pl.ds(off, nr * CH)], osem
                )
                if cond:
                    @pl.when(has_extra)
                    def _(r0=r0, nr=nr, buf=buf):
                        pltpu.async_copy(
                            buf.at[pl.ds(nr * CH, CH)],
                            out_hbm.at[pl.ds((row0 + r0 + nr) * CH, CH)],
                            osem,
                        ).wait()
            for d in out_desc:
                if d is not None:
                    d.wait()

        do(row_hbm, xr_hbm)
        do(col_hbm, xc_hbm)

    return k


@functools.lru_cache(maxsize=None)
def _scatter_add(E, D, N):
    nchk, base, extra = _tile_rows(E)
    maxrows = base + (1 if extra else 0)
    nfullg = base // GRP
    rem = base - nfullg * GRP
    RPT = N // NS
    mesh = plsc.VectorSubcoreMesh(core_axis_name="c", subcore_axis_name="s")

    @functools.partial(
        pl.kernel,
        mesh=mesh,
        out_type=jax.ShapeDtypeStruct((NC, N, D), jnp.float32),
        scratch_types=[
            pltpu.VMEM((maxrows, CH), jnp.int32),
            pltpu.VMEM((GRP * CH, D), jnp.float32),
            pltpu.VMEM((GRP * CH, D), jnp.float32),
            pltpu.VMEM_SHARED((N, D), jnp.float32),
            pltpu.SemaphoreType.DMA,
        ],
        compiler_params=pltpu.CompilerParams(use_tc_tiling_on_sc=False),
    )
    def k(vals_hbm, idx_hbm, zeros_hbm, out_hbm, idx_v, bufa, bufb, acc, lsem):
        cid = lax.axis_index("c")
        sid = lax.axis_index("s")
        wid = sid * NC + cid
        row0 = _row_start(wid, base, extra)
        has_extra = wid < extra
        bufs = (bufa, bufb)

        pltpu.sync_copy(
            zeros_hbm.at[pl.ds(sid * RPT, RPT)], acc.at[pl.ds(sid * RPT, RPT)]
        )
        pltpu.sync_copy(idx_hbm.at[pl.ds(row0, base)], idx_v.at[pl.ds(0, base)])
        if extra:
            @pl.when(has_extra)
            def _():
                pltpu.sync_copy(
                    idx_hbm.at[pl.ds(row0 + base, 1)], idx_v.at[pl.ds(base, 1)]
                )
        plsc.subcore_barrier()

        groups = []
        for g in range(nfullg):
            groups.append((g * GRP, GRP, False))
        if rem or extra:
            groups.append((nfullg * GRP, rem, bool(extra)))

        def load(r0, nr, cond, buf):
            d = pltpu.async_copy(
                vals_hbm.at[pl.ds((row0 + r0) * CH, nr * CH)],
                buf.at[pl.ds(0, nr * CH)],
                lsem,
            )
            dx = [d]
            if cond:
                @pl.when(has_extra)
                def _():
                    pltpu.async_copy(
                        vals_hbm.at[pl.ds((row0 + r0 + nr) * CH, CH)],
                        buf.at[pl.ds(nr * CH, CH)],
                        lsem,
                    ).wait()
            return dx

        descs = {}
        descs[0] = load(*groups[0], bufs[0])
        if len(groups) > 1:
            descs[1] = load(*groups[1], bufs[1])
        for gi, (r0, nr, cond) in enumerate(groups):
            buf = bufs[gi % 2]
            for d in descs.pop(gi):
                d.wait()
            for j in range(nr):
                pltpu.sync_copy(
                    buf.at[pl.ds(j * CH, CH)],
                    acc.at[idx_v.at[r0 + j]],
                    add=True,
                )
            if cond:
                @pl.when(has_extra)
                def _(r0=r0, nr=nr, buf=buf):
                    pltpu.sync_copy(
                        buf.at[pl.ds(nr * CH, CH)],
                        acc.at[idx_v.at[r0 + nr]],
                        add=True,
                    )
            if gi + 2 < len(groups):
                descs[gi + 2] = load(*groups[gi + 2], buf)

        plsc.subcore_barrier()
        pltpu.sync_copy(
            acc.at[pl.ds(sid * RPT, RPT)], out_hbm.at[cid, pl.ds(sid * RPT, RPT)]
        )

    return k


_BE = 4000
_BN = 2000


def _full(shape):
    return pl.BlockSpec(shape, lambda i: (0,) * len(shape))


def _edge_body(xr_r, xc_r, ea_r, A, B, C, b1, W2, b2, D1, Ew, b1n, W2n, b2n,
               ea2_r, h_r):
    f32 = jnp.float32
    t = xr_r[...] @ A[...] + xc_r[...] @ B[...] + ea_r[...] @ C[...] + b1[...]
    t = jnp.maximum(t, 0.0)
    ea2 = jax.lax.dot(t, W2[...], preferred_element_type=f32) + b2[...]
    ea2_r[...] = ea2
    t2 = xr_r[...] @ D1[...] + ea2 @ Ew[...] + b1n[...]
    t2 = jnp.maximum(t2, 0.0)
    h_r[...] = jax.lax.dot(t2, W2n[...], preferred_element_type=f32) + b2n[...]


def _tc_edge(xr, xc, ea, A, B, C, b1, W2, b2, D1, Ew, b1n, W2n, b2n):
    E, Dx = xr.shape
    De = ea.shape[1]
    G = E // _BE
    grid_spec = pl.GridSpec(
        grid=(G,),
        in_specs=[
            pl.BlockSpec((_BE, Dx), lambda i: (i, 0)),
            pl.BlockSpec((_BE, Dx), lambda i: (i, 0)),
            pl.BlockSpec((_BE, De), lambda i: (i, 0)),
            _full(A.shape), _full(B.shape), _full(C.shape), _full(b1.shape),
            _full(W2.shape), _full(b2.shape), _full(D1.shape), _full(Ew.shape),
            _full(b1n.shape), _full(W2n.shape), _full(b2n.shape),
        ],
        out_specs=[
            pl.BlockSpec((_BE, 64), lambda i: (i, 0)),
            pl.BlockSpec((_BE, 64), lambda i: (i, 0)),
        ],
    )
    return pl.pallas_call(
        _edge_body,
        grid_spec=grid_spec,
        out_shape=[
            jax.ShapeDtypeStruct((E, 64), jnp.float32),
            jax.ShapeDtypeStruct((E, 64), jnp.float32),
        ],
        compiler_params=pltpu.CompilerParams(
            dimension_semantics=("arbitrary",)
        ),
    )(xr, xc, ea, A, B, C, b1, W2, b2, D1, Ew, b1n, W2n, b2n)


def _edge_final_body(xr_r, xc_r, ea_r, A, B, C, b1, W2, b2, out_r):
    t = xr_r[...] @ A[...] + xc_r[...] @ B[...] + ea_r[...] @ C[...] + b1[...]
    t = jnp.maximum(t, 0.0)
    z = jax.lax.dot(t, W2[...], preferred_element_type=jnp.float32) + b2[...]
    out_r[...] = jax.nn.sigmoid(z)


def _tc_edge_final(xr, xc, ea, A, B, C, b1, W2, b2):
    E, Dx = xr.shape
    De = ea.shape[1]
    G = E // _BE
    grid_spec = pl.GridSpec(
        grid=(G,),
        in_specs=[
            pl.BlockSpec((_BE, Dx), lambda i: (i, 0)),
            pl.BlockSpec((_BE, Dx), lambda i: (i, 0)),
            pl.BlockSpec((_BE, De), lambda i: (i, 0)),
            _full(A.shape), _full(B.shape), _full(C.shape), _full(b1.shape),
            _full(W2.shape), _full(b2.shape),
        ],
        out_specs=[pl.BlockSpec((_BE, 1), lambda i: (i, 0))],
    )
    return pl.pallas_call(
        _edge_final_body,
        grid_spec=grid_spec,
        out_shape=[jax.ShapeDtypeStruct((E, 1), jnp.float32)],
        compiler_params=pltpu.CompilerParams(
            dimension_semantics=("arbitrary",)
        ),
    )(xr, xc, ea, A, B, C, b1, W2, b2)[0]


def _node_body(x_r, sp_r, cp_r, Wx, Wa, b1, W2, b2, out_r):
    s = sp_r[0] + sp_r[1]
    c = cp_r[0, :, 0:1] + cp_r[1, :, 0:1]
    agg = s / jnp.maximum(c, 1.0)
    t = x_r[...] @ Wx[...] + agg @ Wa[...] + b1[...]
    t = jnp.maximum(t, 0.0)
    out_r[...] = jax.lax.dot(t, W2[...], preferred_element_type=jnp.float32) + b2[...]


def _tc_node(x, spart, cpart, Wx, Wa, b1, W2, b2):
    N, Dx = x.shape
    G = N // _BN
    grid_spec = pl.GridSpec(
        grid=(G,),
        in_specs=[
            pl.BlockSpec((_BN, Dx), lambda i: (i, 0)),
            pl.BlockSpec((NC, _BN, 64), lambda i: (0, i, 0)),
            pl.BlockSpec((NC, _BN, 16), lambda i: (0, i, 0)),
            _full(Wx.shape), _full(Wa.shape), _full(b1.shape),
            _full(W2.shape), _full(b2.shape),
        ],
        out_specs=[pl.BlockSpec((_BN, 64), lambda i: (i, 0))],
    )
    return pl.pallas_call(
        _node_body,
        grid_spec=grid_spec,
        out_shape=[jax.ShapeDtypeStruct((N, 64), jnp.float32)],
        compiler_params=pltpu.CompilerParams(
            dimension_semantics=("arbitrary",)
        ),
    )(x, spart, cpart, Wx, Wa, b1, W2, b2)[0]


def _row(b):
    return b.reshape(1, -1)


def _pad_rows(W, rows_from, n_rows):
    sub = W[rows_from[0]:rows_from[1]]
    return jnp.pad(sub, ((0, n_rows - sub.shape[0]), (0, 0)))


def kernel(x, edge_index, edge_attr, params):
    N, _ = x.shape
    E = edge_attr.shape[0]
    row = edge_index[0].reshape(E // CH, CH)
    col = edge_index[1].reshape(E // CH, CH)

    x16 = jnp.pad(x, ((0, 0), (0, 14)))
    ea16 = jnp.pad(edge_attr, ((0, 0), (0, 12)))
    zeros64 = jnp.zeros((N, 64), jnp.float32)
    zeros16 = jnp.zeros((N, 16), jnp.float32)
    ones16 = jnp.ones((E, 16), jnp.float32)

    cpart = _scatter_add(E, 16, N)(ones16, col, zeros16)

    W1e, b1e, W2e, b2e = params['c1_e']
    W1n, b1n, W2n, b2n = params['c1_n1']
    V1n, c1n, V2n, c2n = params['c1_n2']
    A = _pad_rows(W1e, (0, 2), 16)
    B = _pad_rows(W1e, (2, 4), 16)
    C = _pad_rows(W1e, (4, 8), 16)
    D1 = _pad_rows(W1n, (0, 2), 16)
    Ew = W1n[2:66]
    Wx = _pad_rows(V1n, (0, 2), 16)
    Wa = V1n[2:66]

    xr, xc = _gather2(N, 16, E)(x16, row, col)
    ea, h = _tc_edge(xr, xc, ea16, A, B, C, _row(b1e), W2e, _row(b2e),
                     D1, Ew, _row(b1n), W2n, _row(b2n))
    spart = _scatter_add(E, 64, N)(h, col, zeros64)
    xl = _tc_node(x16, spart, cpart, Wx, Wa, _row(c1n), V2n, _row(c2n))

    for name in ('c2', 'c3', 'c4'):
        W1e, b1e, W2e, b2e = params[name + '_e']
        W1n, b1n, W2n, b2n = params[name + '_n1']
        V1n, c1n, V2n, c2n = params[name + '_n2']
        xr, xc = _gather2(N, 64, E)(xl, row, col)
        ea, h = _tc_edge(xr, xc, ea,
                         W1e[0:64], W1e[64:128], W1e[128:192], _row(b1e),
                         W2e, _row(b2e),
                         W1n[0:64], W1n[64:128], _row(b1n), W2n, _row(b2n))
        spart = _scatter_add(E, 64, N)(h, col, zeros64)
        xl = _tc_node(xl, spart, cpart,
                      V1n[0:64], V1n[64:128], _row(c1n), V2n, _row(c2n))

    W1f, b1f, W2f, b2f = params['c5_e']
    xr, xc = _gather2(N, 64, E)(xl, row, col)
    return _tc_edge_final(xr, xc, ea,
                          W1f[0:64], W1f[64:128], W1f[128:192], _row(b1f),
                          W2f, _row(b2f))


# speedup vs baseline: 2.3343x; 1.0109x over previous
import functools

import jax
import jax.numpy as jnp
from jax import lax
from jax.experimental import pallas as pl
from jax.experimental.pallas import tpu as pltpu
from jax.experimental.pallas import tpu_sc as plsc

NC, NS, L = 2, 16, 16
NW = NC * NS
CH = 128


GRP = 4


def _tile_rows(E):
    nchk = E // CH
    base = nchk // NW
    extra = nchk - base * NW
    return nchk, base, extra


def _row_start(wid, base, extra):
    return jnp.where(
        wid < extra, (base + 1) * wid, extra * (base + 1) + base * (wid - extra)
    )


@functools.lru_cache(maxsize=None)
def _gather2(N, D, E):
    nchk, base, extra = _tile_rows(E)
    maxrows = base + (1 if extra else 0)
    nfullg = base // GRP
    rem = base - nfullg * GRP
    mesh = plsc.VectorSubcoreMesh(core_axis_name="c", subcore_axis_name="s")

    @functools.partial(
        pl.kernel,
        mesh=mesh,
        out_type=(
            jax.ShapeDtypeStruct((E, D), jnp.float32),
            jax.ShapeDtypeStruct((E, D), jnp.float32),
        ),
        scratch_types=[
            pltpu.VMEM((maxrows, CH), jnp.int32),
            pltpu.VMEM((GRP * CH, D), jnp.float32),
            pltpu.VMEM((GRP * CH, D), jnp.float32),
            pltpu.SemaphoreType.DMA,
            pltpu.SemaphoreType.DMA,
        ],
        compiler_params=pltpu.CompilerParams(use_tc_tiling_on_sc=False),
    )
    def k(table_hbm, row_hbm, col_hbm, xr_hbm, xc_hbm, idx_v, bufa, bufb, gsem, osem):
        wid = lax.axis_index("s") * NC + lax.axis_index("c")
        row0 = _row_start(wid, base, extra)
        has_extra = wid < extra
        bufs = (bufa, bufb)

        def do(idx2d_hbm, out_hbm):
            pltpu.sync_copy(
                idx2d_hbm.at[pl.ds(row0, base)], idx_v.at[pl.ds(0, base)]
            )
            if extra:
                @pl.when(has_extra)
                def _():
                    pltpu.sync_copy(
                        idx2d_hbm.at[pl.ds(row0 + base, 1)],
                        idx_v.at[pl.ds(base, 1)],
                    )

            groups = []
            for g in range(nfullg):
                groups.append((g * GRP, GRP, False))
            if rem or extra:
                groups.append((nfullg * GRP, rem, bool(extra)))

            out_desc = [None, None]
            for gi, (r0, nr, cond) in enumerate(groups):
                buf = bufs[gi % 2]
                if out_desc[gi % 2] is not None:
                    out_desc[gi % 2].wait()
                descs = []
                for j in range(nr):
                    descs.append(
                        pltpu.async_copy(
                            table_hbm.at[idx_v.at[r0 + j]],
                            buf.at[pl.ds(j * CH, CH)],
                            gsem,
                        )
                    )
                if cond:
                    @pl.when(has_extra)
                    def _(r0=r0, nr=nr, buf=buf):
                        pltpu.async_copy(
                            table_hbm.at[idx_v.at[r0 + nr]],
                            buf.at[pl.ds(nr * CH, CH)],
                            gsem,
                        ).wait()
                for d in descs:
                    d.wait()
                off = (row0 + r0) * CH
                out_desc[gi % 2] = pltpu.async_copy(
                    buf.at[pl.ds(0, nr * CH)], out_hbm.at[pl.ds(off, nr * CH)], osem
                )
                if cond:
                    @pl.when(has_extra)
                    def _(r0=r0, nr=nr, buf=buf):
                        pltpu.async_copy(
                            buf.at[pl.ds(nr * CH, CH)],
                            out_hbm.at[pl.ds((row0 + r0 + nr) * CH, CH)],
                            osem,
                        ).wait()
            for d in out_desc:
                if d is not None:
                    d.wait()

        do(row_hbm, xr_hbm)
        do(col_hbm, xc_hbm)

    return k


@functools.lru_cache(maxsize=None)
def _scatter_add(E, D, N):
    nchk, base, extra = _tile_rows(E)
    maxrows = base + (1 if extra else 0)
    nfullg = base // GRP
    rem = base - nfullg * GRP
    RPT = N // NS
    mesh = plsc.VectorSubcoreMesh(core_axis_name="c", subcore_axis_name="s")

    @functools.partial(
        pl.kernel,
        mesh=mesh,
        out_type=jax.ShapeDtypeStruct((NC, N, D), jnp.float32),
        scratch_types=[
            pltpu.VMEM((maxrows, CH), jnp.int32),
            pltpu.VMEM((GRP * CH, D), jnp.float32),
            pltpu.VMEM((GRP * CH, D), jnp.float32),
            pltpu.VMEM_SHARED((N, D), jnp.float32),
            pltpu.SemaphoreType.DMA,
        ],
        compiler_params=pltpu.CompilerParams(use_tc_tiling_on_sc=False),
    )
    def k(vals_hbm, idx_hbm, zeros_hbm, out_hbm, idx_v, bufa, bufb, acc, lsem):
        cid = lax.axis_index("c")
        sid = lax.axis_index("s")
        wid = sid * NC + cid
        row0 = _row_start(wid, base, extra)
        has_extra = wid < extra
        bufs = (bufa, bufb)

        pltpu.sync_copy(
            zeros_hbm.at[pl.ds(sid * RPT, RPT)], acc.at[pl.ds(sid * RPT, RPT)]
        )
        pltpu.sync_copy(idx_hbm.at[pl.ds(row0, base)], idx_v.at[pl.ds(0, base)])
        if extra:
            @pl.when(has_extra)
            def _():
                pltpu.sync_copy(
                    idx_hbm.at[pl.ds(row0 + base, 1)], idx_v.at[pl.ds(base, 1)]
                )
        plsc.subcore_barrier()

        groups = []
        for g in range(nfullg):
            groups.append((g * GRP, GRP, False))
        if rem or extra:
            groups.append((nfullg * GRP, rem, bool(extra)))

        def load(r0, nr, cond, buf):
            d = pltpu.async_copy(
                vals_hbm.at[pl.ds((row0 + r0) * CH, nr * CH)],
                buf.at[pl.ds(0, nr * CH)],
                lsem,
            )
            dx = [d]
            if cond:
                @pl.when(has_extra)
                def _():
                    pltpu.async_copy(
                        vals_hbm.at[pl.ds((row0 + r0 + nr) * CH, CH)],
                        buf.at[pl.ds(nr * CH, CH)],
                        lsem,
                    ).wait()
            return dx

        descs = {}
        descs[0] = load(*groups[0], bufs[0])
        if len(groups) > 1:
            descs[1] = load(*groups[1], bufs[1])
        for gi, (r0, nr, cond) in enumerate(groups):
            buf = bufs[gi % 2]
            for d in descs.pop(gi):
                d.wait()
            for j in range(nr):
                pltpu.sync_copy(
                    buf.at[pl.ds(j * CH, CH)],
                    acc.at[idx_v.at[r0 + j]],
                    add=True,
                )
            if cond:
                @pl.when(has_extra)
                def _(r0=r0, nr=nr, buf=buf):
                    pltpu.sync_copy(
                        buf.at[pl.ds(nr * CH, CH)],
                        acc.at[idx_v.at[r0 + nr]],
                        add=True,
                    )
            if gi + 2 < len(groups):
                descs[gi + 2] = load(*groups[gi + 2], buf)

        plsc.subcore_barrier()
        pltpu.sync_copy(
            acc.at[pl.ds(sid * RPT, RPT)], out_hbm.at[cid, pl.ds(sid * RPT, RPT)]
        )

    return k


_BE = 4000
_BN = 2000


def _full(shape):
    return pl.BlockSpec(shape, lambda i: (0,) * len(shape))


def _edge_body(hw, xr_r, xc_r, ea_r, G1, bz1, G2, be2, bh, W2n, b2n,
               ea2_r, h_r):
    bf = jnp.bfloat16
    f32 = jnp.float32
    xcat = jnp.concatenate(
        [xr_r[...].astype(bf), xc_r[...].astype(bf), ea_r[...].astype(bf)],
        axis=1,
    )
    Z = jax.lax.dot(xcat, G1[...], preferred_element_type=f32) + bz1[...]
    t = jnp.maximum(Z[:, 0:64], 0.0).astype(bf)
    P = jax.lax.dot(t, G2[...], preferred_element_type=f32)
    ea2_r[...] = P[:, 0:64] + be2[...]
    t2 = jnp.maximum(P[:, 64:128] + Z[:, 64:128] + bh[...], 0.0).astype(bf)
    h = jax.lax.dot(t2, W2n[...], preferred_element_type=f32) + b2n[...]
    if hw > 64:
        h = jnp.concatenate([h, jnp.ones((h.shape[0], hw - 64), f32)], axis=1)
    h_r[...] = h


def _tc_edge(xr, xc, ea, G1, bz1, G2, be2, bh, W2n, b2n, hw=64):
    E, Dx = xr.shape
    De = ea.shape[1]
    G = E // _BE
    grid_spec = pl.GridSpec(
        grid=(G,),
        in_specs=[
            pl.BlockSpec((_BE, Dx), lambda i: (i, 0)),
            pl.BlockSpec((_BE, Dx), lambda i: (i, 0)),
            pl.BlockSpec((_BE, De), lambda i: (i, 0)),
            _full(G1.shape), _full(bz1.shape), _full(G2.shape),
            _full(be2.shape), _full(bh.shape), _full(W2n.shape),
            _full(b2n.shape),
        ],
        out_specs=[
            pl.BlockSpec((_BE, 64), lambda i: (i, 0)),
            pl.BlockSpec((_BE, hw), lambda i: (i, 0)),
        ],
    )
    return pl.pallas_call(
        functools.partial(_edge_body, hw),
        grid_spec=grid_spec,
        out_shape=[
            jax.ShapeDtypeStruct((E, 64), jnp.float32),
            jax.ShapeDtypeStruct((E, hw), jnp.float32),
        ],
        compiler_params=pltpu.CompilerParams(
            dimension_semantics=("arbitrary",)
        ),
    )(xr, xc, ea, G1, bz1, G2, be2, bh, W2n, b2n)


def _edge_final_body(xr_r, xc_r, ea_r, W1, b1, W2, b2, out_r):
    bf = jnp.bfloat16
    xcat = jnp.concatenate(
        [xr_r[...].astype(bf), xc_r[...].astype(bf), ea_r[...].astype(bf)],
        axis=1,
    )
    t = jax.lax.dot(xcat, W1[...], preferred_element_type=jnp.float32) + b1[...]
    t = jnp.maximum(t, 0.0).astype(bf)
    z = jax.lax.dot(t, W2[...], preferred_element_type=jnp.float32) + b2[...]
    out_r[...] = jax.nn.sigmoid(z)


def _tc_edge_final(xr, xc, ea, W1, b1, W2, b2):
    E, Dx = xr.shape
    De = ea.shape[1]
    G = E // _BE
    grid_spec = pl.GridSpec(
        grid=(G,),
        in_specs=[
            pl.BlockSpec((_BE, Dx), lambda i: (i, 0)),
            pl.BlockSpec((_BE, Dx), lambda i: (i, 0)),
            pl.BlockSpec((_BE, De), lambda i: (i, 0)),
            _full(W1.shape), _full(b1.shape),
            _full(W2.shape), _full(b2.shape),
        ],
        out_specs=[pl.BlockSpec((_BE, 1), lambda i: (i, 0))],
    )
    return pl.pallas_call(
        _edge_final_body,
        grid_spec=grid_spec,
        out_shape=[jax.ShapeDtypeStruct((E, 1), jnp.float32)],
        compiler_params=pltpu.CompilerParams(
            dimension_semantics=("arbitrary",)
        ),
    )(xr, xc, ea, W1, b1, W2, b2)[0]


def _node_body(ds, x_r, sp_r, cp_r, V1, b1, W2, b2, out_r):
    bf = jnp.bfloat16
    if ds > 64:
        s = sp_r[0, :, 0:64] + sp_r[1, :, 0:64]
        c = sp_r[0, :, 64:65] + sp_r[1, :, 64:65]
    else:
        s = sp_r[0] + sp_r[1]
        c = cp_r[0, :, 0:1] + cp_r[1, :, 0:1]
    agg = s / jnp.maximum(c, 1.0)
    xcat = jnp.concatenate([x_r[...].astype(bf), agg.astype(bf)], axis=1)
    t = jax.lax.dot(xcat, V1[...], preferred_element_type=jnp.float32) + b1[...]
    t = jnp.maximum(t, 0.0).astype(bf)
    out_r[...] = jax.lax.dot(t, W2[...], preferred_element_type=jnp.float32) + b2[...]


def _tc_node(x, spart, cpart, V1, b1, W2, b2):
    N, Dx = x.shape
    ds = spart.shape[2]
    G = N // _BN
    grid_spec = pl.GridSpec(
        grid=(G,),
        in_specs=[
            pl.BlockSpec((_BN, Dx), lambda i: (i, 0)),
            pl.BlockSpec((NC, _BN, ds), lambda i: (0, i, 0)),
            pl.BlockSpec((NC, _BN, 16), lambda i: (0, i, 0)),
            _full(V1.shape), _full(b1.shape),
            _full(W2.shape), _full(b2.shape),
        ],
        out_specs=[pl.BlockSpec((_BN, 64), lambda i: (i, 0))],
    )
    return pl.pallas_call(
        functools.partial(_node_body, ds),
        grid_spec=grid_spec,
        out_shape=[jax.ShapeDtypeStruct((N, 64), jnp.float32)],
        compiler_params=pltpu.CompilerParams(
            dimension_semantics=("arbitrary",)
        ),
    )(x, spart, cpart, V1, b1, W2, b2)[0]


def _row(b):
    return b.reshape(1, -1)


def _pad_rows(W, rows_from, n_rows):
    sub = W[rows_from[0]:rows_from[1]]
    return jnp.pad(sub, ((0, n_rows - sub.shape[0]), (0, 0)))


def _prep_edge(pe, pn1, dx_raw, dx):
    bf = jnp.bfloat16
    W1e, b1e, W2e, b2e = pe
    W1n, b1n, W2n, b2n = pn1
    de_raw = W1e.shape[0] - 2 * dx_raw
    if dx_raw == dx:
        A, B, C = W1e[0:dx], W1e[dx:2 * dx], W1e[2 * dx:]
    else:
        A = _pad_rows(W1e, (0, dx_raw), dx)
        B = _pad_rows(W1e, (dx_raw, 2 * dx_raw), dx)
        C = _pad_rows(W1e, (2 * dx_raw, 2 * dx_raw + de_raw), dx)
    D1 = W1n[0:dx_raw]
    if dx_raw != dx:
        D1 = _pad_rows(W1n, (0, dx_raw), dx)
    Ew = W1n[dx_raw:dx_raw + 64]
    G1 = jnp.concatenate(
        [jnp.concatenate([A, B, C], axis=0),
         jnp.concatenate([D1, jnp.zeros((2 * dx, 64), jnp.float32)], axis=0)],
        axis=1).astype(bf)
    bz1 = jnp.concatenate([b1e, jnp.zeros((64,), jnp.float32)]).reshape(1, 128)
    W2eEw = W2e @ Ew
    G2 = jnp.concatenate([W2e, W2eEw], axis=1).astype(bf)
    bh = _row(b2e @ Ew + b1n)
    return G1, bz1, G2, _row(b2e), bh, W2n.astype(bf), _row(b2n)


def _prep_node(pn2, dx_raw, dx):
    bf = jnp.bfloat16
    V1n, c1n, V2n, c2n = pn2
    Wx = V1n[0:dx_raw]
    if dx_raw != dx:
        Wx = _pad_rows(V1n, (0, dx_raw), dx)
    Wa = V1n[dx_raw:dx_raw + 64]
    V1 = jnp.concatenate([Wx, Wa], axis=0).astype(bf)
    return V1, _row(c1n), V2n.astype(bf), _row(c2n)


def kernel(x, edge_index, edge_attr, params):
    N, _ = x.shape
    E = edge_attr.shape[0]
    row = edge_index[0].reshape(E // CH, CH)
    col = edge_index[1].reshape(E // CH, CH)

    x16 = jnp.pad(x, ((0, 0), (0, 14)))
    ea16 = jnp.pad(edge_attr, ((0, 0), (0, 12)))
    zeros64 = jnp.zeros((N, 64), jnp.float32)
    zeros16 = jnp.zeros((N, 16), jnp.float32)
    ones16 = jnp.ones((E, 16), jnp.float32)

    cpart = _scatter_add(E, 16, N)(ones16, col, zeros16)

    ew = _prep_edge(params['c1_e'], params['c1_n1'], 2, 16)
    xr, xc = _gather2(N, 16, E)(x16, row, col)
    ea, h = _tc_edge(xr, xc, ea16, *ew)
    spart = _scatter_add(E, 64, N)(h, col, zeros64)
    nw = _prep_node(params['c1_n2'], 2, 16)
    xl = _tc_node(x16, spart, cpart, *nw)

    for name in ('c2', 'c3', 'c4'):
        ew = _prep_edge(params[name + '_e'], params[name + '_n1'], 64, 64)
        xr, xc = _gather2(N, 64, E)(xl, row, col)
        ea, h = _tc_edge(xr, xc, ea, *ew)
        spart = _scatter_add(E, 64, N)(h, col, zeros64)
        nw = _prep_node(params[name + '_n2'], 64, 64)
        xl = _tc_node(xl, spart, cpart, *nw)

    W1f, b1f, W2f, b2f = params['c5_e']
    xr, xc = _gather2(N, 64, E)(xl, row, col)
    return _tc_edge_final(xr, xc, ea, W1f.astype(jnp.bfloat16), _row(b1f),
                          W2f.astype(jnp.bfloat16), _row(b2f).reshape(1, 1))
